# initial kernel scaffold (unmeasured)
import jax
import jax.numpy as jnp
from jax import lax
from jax.experimental import pallas as pl
from jax.experimental.pallas import tpu as pltpu

N_DEV = 16
B, SQ, D = 2, 256, 768
HQ_LOC, DH = 8, 64
HKV_LOC = 2
ROWS = B * SQ
CHUNK = ROWS // N_DEV


def kernel(x, Wq, Wo, K_ext, V_ext):
    idx = lax.axis_index("i")
    Ks = lax.dynamic_slice_in_dim(K_ext, idx * HKV_LOC, HKV_LOC, axis=2)
    Vs = lax.dynamic_slice_in_dim(V_ext, idx * HKV_LOC, HKV_LOC, axis=2)

    def body(x_ref, wq_ref, wo_ref, k_ref, v_ref, out_ref,
             pacc, comm, sem1, ssem1, sem2, ssem2):
        me = lax.axis_index("i")

        def rdma(src, dst, ssem, rsem, dev):
            return pltpu.make_async_remote_copy(
                src_ref=src, dst_ref=dst,
                send_sem=ssem, recv_sem=rsem,
                device_id=(dev,), device_id_type=pl.DeviceIdType.MESH,
            )

        wq = wq_ref[...].astype(jnp.bfloat16)
        wo = wo_ref[...].astype(jnp.bfloat16)
        for b in range(B):
            xb = x_ref[b].astype(jnp.bfloat16)
            qb = jnp.dot(xb, wq, preferred_element_type=jnp.float32)
            qb = (qb * 0.125).astype(jnp.bfloat16)
            outs = []
            for h in range(HQ_LOC):
                g = h // 4
                k = k_ref[b, :, g, :].astype(jnp.bfloat16)
                v = v_ref[b, :, g, :].astype(jnp.bfloat16)
                qh = qb[:, h * DH:(h + 1) * DH]
                s = lax.dot_general(
                    qh, k, (((1,), (1,)), ((), ())),
                    preferred_element_type=jnp.float32)
                m = jnp.max(s, axis=1, keepdims=True)
                p = jnp.exp(s - m)
                l = jnp.sum(p, axis=1, keepdims=True)
                o = jnp.dot(p.astype(jnp.bfloat16), v,
                            preferred_element_type=jnp.float32) / l
                outs.append(o.astype(jnp.bfloat16))
            ob = jnp.concatenate(outs, axis=1)
            pacc[pl.ds(b * SQ, SQ), :] = jnp.dot(
                ob, wo, preferred_element_type=jnp.float32)

        comm[pl.ds(me * CHUNK, CHUNK), :] = pacc[pl.ds(me * CHUNK, CHUNK), :]

        for c in range(N_DEV):
            @pl.when(me != c)
            def _(c=c):
                rdma(pacc.at[pl.ds(c * CHUNK, CHUNK)],
                     comm.at[pl.ds(me * CHUNK, CHUNK)],
                     ssem1.at[c], sem1.at[me], c).start()

        for d in range(N_DEV):
            @pl.when(me != d)
            def _(d=d):
                rdma(comm.at[pl.ds(d * CHUNK, CHUNK)],
                     comm.at[pl.ds(d * CHUNK, CHUNK)],
                     ssem1.at[d], sem1.at[d], d).wait_recv()

        red = jnp.sum(comm[...].reshape(N_DEV, CHUNK, D), axis=0)
        out_ref[pl.ds(me * CHUNK, CHUNK), :] = red

        for c in range(N_DEV):
            @pl.when(me != c)
            def _(c=c):
                rdma(out_ref.at[pl.ds(me * CHUNK, CHUNK)],
                     out_ref.at[pl.ds(me * CHUNK, CHUNK)],
                     ssem2.at[c], sem2.at[me], c).start()

        for c in range(N_DEV):
            @pl.when(me != c)
            def _(c=c):
                rdma(pacc.at[pl.ds(c * CHUNK, CHUNK)],
                     comm.at[pl.ds(c * CHUNK, CHUNK)],
                     ssem1.at[c], sem1.at[me], c).wait_send()

        for d in range(N_DEV):
            @pl.when(me != d)
            def _(d=d):
                rdma(out_ref.at[pl.ds(d * CHUNK, CHUNK)],
                     out_ref.at[pl.ds(d * CHUNK, CHUNK)],
                     ssem2.at[d], sem2.at[d], d).wait_recv()

        for c in range(N_DEV):
            @pl.when(me != c)
            def _(c=c):
                rdma(out_ref.at[pl.ds(me * CHUNK, CHUNK)],
                     out_ref.at[pl.ds(me * CHUNK, CHUNK)],
                     ssem2.at[c], sem2.at[me], c).wait_send()

    out = pl.pallas_call(
        body,
        out_shape=jax.ShapeDtypeStruct((ROWS, D), jnp.float32),
        in_specs=[pl.BlockSpec(memory_space=pltpu.VMEM)] * 5,
        out_specs=pl.BlockSpec(memory_space=pltpu.VMEM),
        scratch_shapes=[
            pltpu.VMEM((ROWS, D), jnp.float32),
            pltpu.VMEM((ROWS, D), jnp.float32),
            pltpu.SemaphoreType.DMA((N_DEV,)),
            pltpu.SemaphoreType.DMA((N_DEV,)),
            pltpu.SemaphoreType.DMA((N_DEV,)),
            pltpu.SemaphoreType.DMA((N_DEV,)),
        ],
        compiler_params=pltpu.CompilerParams(collective_id=0),
    )(x, Wq, Wo, Ks, Vs)
    return out.reshape(B, SQ, D)


# baseline (device time: 57060 ns/iter reference)
import jax
import jax.numpy as jnp
from jax import lax
from jax.experimental import pallas as pl
from jax.experimental.pallas import tpu as pltpu

N_DEV = 16
B, SQ, D = 2, 256, 768
HQ_LOC, DH = 8, 64
HKV_LOC = 2
ROWS = B * SQ
CHUNK = ROWS // N_DEV


def kernel(x, Wq, Wo, K_ext, V_ext):
    idx = lax.axis_index("i")
    Ks = lax.dynamic_slice_in_dim(K_ext, idx * HKV_LOC, HKV_LOC, axis=2)
    Vs = lax.dynamic_slice_in_dim(V_ext, idx * HKV_LOC, HKV_LOC, axis=2)

    def body(x_ref, wq_ref, wo_ref, k_ref, v_ref, out_ref,
             pacc, comm, sem1, ssem1, sem2, ssem2):
        me = lax.axis_index("i")

        def rdma(src, dst, ssem, rsem, dev):
            return pltpu.make_async_remote_copy(
                src_ref=src, dst_ref=dst,
                send_sem=ssem, recv_sem=rsem,
                device_id=(dev,), device_id_type=pl.DeviceIdType.MESH,
            )

        wq = wq_ref[...].astype(jnp.bfloat16)
        wo = wo_ref[...].astype(jnp.bfloat16)
        for b in range(B):
            xb = x_ref[b].astype(jnp.bfloat16)
            qb = jnp.dot(xb, wq, preferred_element_type=jnp.float32)
            qb = (qb * 0.125).astype(jnp.bfloat16)
            outs = []
            for h in range(HQ_LOC):
                g = h // 4
                k = k_ref[b, :, g, :].astype(jnp.bfloat16)
                v = v_ref[b, :, g, :].astype(jnp.bfloat16)
                qh = qb[:, h * DH:(h + 1) * DH]
                s = lax.dot_general(
                    qh, k, (((1,), (1,)), ((), ())),
                    preferred_element_type=jnp.float32)
                m = jnp.max(s, axis=1, keepdims=True)
                p = jnp.exp(s - m)
                l = jnp.sum(p, axis=1, keepdims=True)
                o = jnp.dot(p.astype(jnp.bfloat16), v,
                            preferred_element_type=jnp.float32) / l
                outs.append(o.astype(jnp.bfloat16))
            ob = jnp.concatenate(outs, axis=1)
            pacc[pl.ds(b * SQ, SQ), :] = jnp.dot(
                ob, wo, preferred_element_type=jnp.float32)

        comm[pl.ds(me * CHUNK, CHUNK), :] = pacc[pl.ds(me * CHUNK, CHUNK), :]

        for c in range(N_DEV):
            @pl.when(me != c)
            def _(c=c):
                rdma(pacc.at[pl.ds(c * CHUNK, CHUNK)],
                     comm.at[pl.ds(me * CHUNK, CHUNK)],
                     ssem1.at[c], sem1.at[me], c).start()

        for d in range(N_DEV):
            @pl.when(me != d)
            def _(d=d):
                rdma(comm.at[pl.ds(d * CHUNK, CHUNK)],
                     comm.at[pl.ds(d * CHUNK, CHUNK)],
                     ssem1.at[d], sem1.at[d], d).wait_recv()

        red = jnp.sum(comm[...].reshape(N_DEV, CHUNK, D), axis=0)
        out_ref[pl.ds(me * CHUNK, CHUNK), :] = red

        for c in range(N_DEV):
            @pl.when(me != c)
            def _(c=c):
                rdma(out_ref.at[pl.ds(me * CHUNK, CHUNK)],
                     out_ref.at[pl.ds(me * CHUNK, CHUNK)],
                     ssem2.at[c], sem2.at[me], c).start()

        for c in range(N_DEV):
            @pl.when(me != c)
            def _(c=c):
                rdma(pacc.at[pl.ds(c * CHUNK, CHUNK)],
                     comm.at[pl.ds(c * CHUNK, CHUNK)],
                     ssem1.at[c], sem1.at[me], c).wait_send()

        for d in range(N_DEV):
            @pl.when(me != d)
            def _(d=d):
                rdma(out_ref.at[pl.ds(d * CHUNK, CHUNK)],
                     out_ref.at[pl.ds(d * CHUNK, CHUNK)],
                     ssem2.at[d], sem2.at[d], d).wait_recv()

        for c in range(N_DEV):
            @pl.when(me != c)
            def _(c=c):
                rdma(out_ref.at[pl.ds(me * CHUNK, CHUNK)],
                     out_ref.at[pl.ds(me * CHUNK, CHUNK)],
                     ssem2.at[c], sem2.at[me], c).wait_send()

    out = pl.pallas_call(
        body,
        out_shape=jax.ShapeDtypeStruct((ROWS, D), jnp.float32),
        in_specs=[pl.BlockSpec(memory_space=pltpu.VMEM)] * 5,
        out_specs=pl.BlockSpec(memory_space=pltpu.VMEM),
        scratch_shapes=[
            pltpu.VMEM((ROWS, D), jnp.float32),
            pltpu.VMEM((ROWS, D), jnp.float32),
            pltpu.SemaphoreType.DMA((N_DEV,)),
            pltpu.SemaphoreType.DMA((N_DEV,)),
            pltpu.SemaphoreType.DMA((N_DEV,)),
            pltpu.SemaphoreType.DMA((N_DEV,)),
        ],
    )(x, Wq, Wo, Ks, Vs)
    return out.reshape(B, SQ, D)


# device time: 44955 ns/iter; 1.2693x vs baseline; 1.2693x over previous
import jax
import jax.numpy as jnp
from jax import lax
from jax.experimental import pallas as pl
from jax.experimental.pallas import tpu as pltpu

N_DEV = 16
B, SQ, D = 2, 256, 768
HQ_LOC, DH = 8, 64
HKV_LOC = 2
ROWS = B * SQ
CHUNK = ROWS // N_DEV


def kernel(x, Wq, Wo, K_ext, V_ext):
    idx = lax.axis_index("i")
    Ks = lax.dynamic_slice_in_dim(K_ext, idx * HKV_LOC, HKV_LOC, axis=2)
    Vs = lax.dynamic_slice_in_dim(V_ext, idx * HKV_LOC, HKV_LOC, axis=2)

    def body(x_ref, wq_ref, wo_ref, k_ref, v_ref, out_ref,
             stage, comm, sem1, ssem1, sem2, ssem2):
        me = lax.axis_index("i")

        def rdma(src, dst, ssem, rsem, dev):
            return pltpu.make_async_remote_copy(
                src_ref=src, dst_ref=dst,
                send_sem=ssem, recv_sem=rsem,
                device_id=(dev,), device_id_type=pl.DeviceIdType.MESH,
            )

        wq = wq_ref[...].astype(jnp.bfloat16)
        wo = wo_ref[...].astype(jnp.bfloat16)
        for b in range(B):
            xb = x_ref[b].astype(jnp.bfloat16)
            qb = jnp.dot(xb, wq, preferred_element_type=jnp.float32)
            qb = (qb * 0.125).astype(jnp.bfloat16)
            outs = []
            for h in range(HQ_LOC):
                g = h // 4
                k = k_ref[b, :, g, :].astype(jnp.bfloat16)
                v = v_ref[b, :, g, :].astype(jnp.bfloat16)
                qh = qb[:, h * DH:(h + 1) * DH]
                s = lax.dot_general(
                    qh, k, (((1,), (1,)), ((), ())),
                    preferred_element_type=jnp.float32)
                m = jnp.max(s, axis=1, keepdims=True)
                p = jnp.exp(s - m)
                l = jnp.sum(p, axis=1, keepdims=True)
                o = jnp.dot(p.astype(jnp.bfloat16), v,
                            preferred_element_type=jnp.float32) / l
                outs.append(o.astype(jnp.bfloat16))
            ob = jnp.concatenate(outs, axis=1)
            pb = jnp.dot(ob, wo, preferred_element_type=jnp.float32)
            stage[pl.ds(b * SQ, SQ), :] = pb.astype(jnp.bfloat16)

        comm[pl.ds(me * CHUNK, CHUNK), :] = stage[pl.ds(me * CHUNK, CHUNK), :]

        for c in range(N_DEV):
            @pl.when(me != c)
            def _(c=c):
                rdma(stage.at[pl.ds(c * CHUNK, CHUNK)],
                     comm.at[pl.ds(me * CHUNK, CHUNK)],
                     ssem1.at[c], sem1.at[me], c).start()

        for d in range(N_DEV):
            @pl.when(me != d)
            def _(d=d):
                rdma(comm.at[pl.ds(d * CHUNK, CHUNK)],
                     comm.at[pl.ds(d * CHUNK, CHUNK)],
                     ssem1.at[d], sem1.at[d], d).wait_recv()

        red = jnp.sum(
            comm[...].reshape(N_DEV, CHUNK, D).astype(jnp.float32), axis=0)
        out_ref[pl.ds(me * CHUNK, CHUNK), :] = red.astype(jnp.bfloat16)

        for c in range(N_DEV):
            @pl.when(me != c)
            def _(c=c):
                rdma(out_ref.at[pl.ds(me * CHUNK, CHUNK)],
                     out_ref.at[pl.ds(me * CHUNK, CHUNK)],
                     ssem2.at[c], sem2.at[me], c).start()

        for c in range(N_DEV):
            @pl.when(me != c)
            def _(c=c):
                rdma(stage.at[pl.ds(c * CHUNK, CHUNK)],
                     comm.at[pl.ds(c * CHUNK, CHUNK)],
                     ssem1.at[c], sem1.at[me], c).wait_send()

        for d in range(N_DEV):
            @pl.when(me != d)
            def _(d=d):
                rdma(out_ref.at[pl.ds(d * CHUNK, CHUNK)],
                     out_ref.at[pl.ds(d * CHUNK, CHUNK)],
                     ssem2.at[d], sem2.at[d], d).wait_recv()

        for c in range(N_DEV):
            @pl.when(me != c)
            def _(c=c):
                rdma(out_ref.at[pl.ds(me * CHUNK, CHUNK)],
                     out_ref.at[pl.ds(me * CHUNK, CHUNK)],
                     ssem2.at[c], sem2.at[me], c).wait_send()

    out = pl.pallas_call(
        body,
        out_shape=jax.ShapeDtypeStruct((ROWS, D), jnp.bfloat16),
        in_specs=[pl.BlockSpec(memory_space=pltpu.VMEM)] * 5,
        out_specs=pl.BlockSpec(memory_space=pltpu.VMEM),
        scratch_shapes=[
            pltpu.VMEM((ROWS, D), jnp.bfloat16),
            pltpu.VMEM((ROWS, D), jnp.bfloat16),
            pltpu.SemaphoreType.DMA((N_DEV,)),
            pltpu.SemaphoreType.DMA((N_DEV,)),
            pltpu.SemaphoreType.DMA((N_DEV,)),
            pltpu.SemaphoreType.DMA((N_DEV,)),
        ],
    )(x, Wq, Wo, Ks, Vs)
    return out.reshape(B, SQ, D)
